# baseline (speedup 1.0000x reference)
import functools

import jax
import jax.numpy as jnp
from jax import lax
from jax.experimental import pallas as pl
from jax.experimental.pallas import tpu as pltpu
from jax.experimental.pallas import tpu_sc as plsc

_BATCH = 16384
_NP = 20
_D = 32
_B = _BATCH * _NP
_NC = 2
_NS = 16
_NW = _NC * _NS
_BPW = _B // _NW
_GROW = 128
_GPC = 5
_CROW = _GROW * _GPC
_CB = _CROW // _NP
_NCHUNK = _BPW // _CROW
_BPWB = _BPW // _NP

_TB = 128
_TD = 8
_NTI = _D // _TD
_NTJ = _BATCH // _TB
_RPT = _TB * _NP * _D // 128

_mesh = plsc.VectorSubcoreMesh(
    core_axis_name="c", subcore_axis_name="s",
    num_cores=_NC, num_subcores=_NS)


@functools.partial(
    pl.kernel,
    out_type=jax.ShapeDtypeStruct((_BATCH, _NP, _D), jnp.float32),
    mesh=_mesh,
    compiler_params=pltpu.CompilerParams(use_tc_tiling_on_sc=False),
    scratch_types=[
        pltpu.VMEM((_BPW,), jnp.int32),
        pltpu.VMEM((_CROW, _D), jnp.float32),
        pltpu.VMEM((_CROW, _D), jnp.float32),
        pltpu.SemaphoreType.DMA,
        pltpu.SemaphoreType.DMA,
        pltpu.SemaphoreType.DMA,
        pltpu.SemaphoreType.DMA,
    ],
)
def _gather_kernel(idx_hbm, table_hbm, out_hbm, idx_v, rows0, rows1,
                   gsem0, gsem1, fsem0, fsem1):
    wid = lax.axis_index("s") * _NC + lax.axis_index("c")
    bbase = wid * _BPWB
    pltpu.sync_copy(idx_hbm.at[pl.ds(wid * _BPW, _BPW)], idx_v)

    def fire_gather(k, rows, sem):
        for g in range(_GPC):
            pltpu.async_copy(
                table_hbm.at[idx_v.at[pl.ds(k * _CROW + g * _GROW, _GROW)]],
                rows.at[pl.ds(g * _GROW, _GROW)], sem)

    def drain_gather(k, rows, sem):
        for g in range(_GPC):
            pltpu.make_async_copy(
                table_hbm.at[idx_v.at[pl.ds(k * _CROW + g * _GROW, _GROW)]],
                rows.at[pl.ds(g * _GROW, _GROW)], sem).wait()

    def fire_flush(k, rows, sem):
        for e in range(_CB):
            pltpu.async_copy(rows.at[pl.ds(e * _NP, _NP)],
                             out_hbm.at[bbase + k * _CB + e], sem)

    def drain_flush(k, rows, sem):
        for e in range(_CB):
            pltpu.make_async_copy(rows.at[pl.ds(e * _NP, _NP)],
                                  out_hbm.at[bbase + k * _CB + e],
                                  sem).wait()

    fire_gather(0, rows0, gsem0)

    @pl.loop(0, _NCHUNK, step=2)
    def _body(k):
        fire_gather(k + 1, rows1, gsem1)
        drain_gather(k, rows0, gsem0)
        fire_flush(k, rows0, fsem0)
        drain_gather(k + 1, rows1, gsem1)
        fire_flush(k + 1, rows1, fsem1)
        drain_flush(k, rows0, fsem0)

        @pl.when(k + 2 < _NCHUNK)
        def _():
            fire_gather(k + 2, rows0, gsem0)

        drain_flush(k + 1, rows1, fsem1)


def _transpose_body(in_ref, out_ref):
    x = in_ref[...]
    y = x.reshape(_TB, _NP // 4, 4, _D)
    z = jnp.transpose(y, (1, 2, 3, 0))
    out_ref[...] = z.reshape(_NP, _NTI, 1, _TD, _TB)


_transpose_kernel = pl.pallas_call(
    _transpose_body,
    grid=(_NTJ,),
    in_specs=[pl.BlockSpec((_RPT, 128), lambda tj: (tj, 0))],
    out_specs=pl.BlockSpec((_NP, _NTI, 1, _TD, _TB),
                           lambda tj: (0, 0, tj, 0, 0)),
    out_shape=jax.ShapeDtypeStruct((_NP, _NTI, _NTJ, _TD, _TB),
                                   jnp.float32),
)


def kernel(species_ids, table):
    flat3 = _gather_kernel(species_ids.reshape(-1).astype(jnp.int32), table)
    in2d = flat3.reshape(_B * _D // 128, 128)
    out5 = _transpose_kernel(in2d)
    return out5.transpose(2, 4, 0, 1, 3).reshape(_BATCH, _NP, _D)


# speedup vs baseline: 1.7817x; 1.7817x over previous
import functools

import jax
import jax.numpy as jnp
from jax import lax
from jax.experimental import pallas as pl
from jax.experimental.pallas import tpu as pltpu
from jax.experimental.pallas import tpu_sc as plsc

_BATCH = 16384
_NP = 20
_D = 32
_B = _BATCH * _NP
_NC = 2
_NS = 16
_NW = _NC * _NS
_TB = 128
_NTJ = _BATCH // _TB
_NQ = 5
_PG = _NP // _NQ
_UROW = _TB * _PG
_GROW = 128
_GPU = _UROW // _GROW
_UPW = _NTJ * _NQ // _NW
_BPW = _UPW * _UROW
_TD = 8
_NTI = _D // _TD

_mesh = plsc.VectorSubcoreMesh(
    core_axis_name="c", subcore_axis_name="s",
    num_cores=_NC, num_subcores=_NS)


@functools.partial(
    pl.kernel,
    out_type=jax.ShapeDtypeStruct((_NTJ * _NQ, _UROW, _D), jnp.float32),
    mesh=_mesh,
    compiler_params=pltpu.CompilerParams(use_tc_tiling_on_sc=False),
    scratch_types=[
        pltpu.VMEM((_BPW,), jnp.int32),
        pltpu.VMEM((_UROW, _D), jnp.float32),
        pltpu.VMEM((_UROW, _D), jnp.float32),
        pltpu.SemaphoreType.DMA,
        pltpu.SemaphoreType.DMA,
        pltpu.SemaphoreType.DMA,
        pltpu.SemaphoreType.DMA,
    ],
)
def _gather_kernel(idx_hbm, table_hbm, out_hbm, idx_v, rowsa, rowsb,
                   gsema, gsemb, fsema, fsemb):
    wid = lax.axis_index("s") * _NC + lax.axis_index("c")
    pltpu.sync_copy(idx_hbm.at[pl.ds(wid * _BPW, _BPW)], idx_v)

    def fire_gathers(u, rows, sem):
        for g in range(_GPU):
            pltpu.async_copy(
                table_hbm.at[idx_v.at[pl.ds(u * _UROW + g * _GROW, _GROW)]],
                rows.at[pl.ds(g * _GROW, _GROW)], sem)

    def drain_gathers(u, rows, sem):
        for g in range(_GPU):
            pltpu.make_async_copy(
                table_hbm.at[idx_v.at[pl.ds(u * _UROW + g * _GROW, _GROW)]],
                rows.at[pl.ds(g * _GROW, _GROW)], sem).wait()

    def fire_flush(u, rows, sem):
        pltpu.async_copy(rows, out_hbm.at[wid * _UPW + u], sem)

    def drain_flush(u, rows, sem):
        pltpu.make_async_copy(rows, out_hbm.at[wid * _UPW + u], sem).wait()

    fire_gathers(0, rowsa, gsema)

    @pl.loop(0, _UPW, step=2)
    def _body(u):
        fire_gathers(u + 1, rowsb, gsemb)
        drain_gathers(u, rowsa, gsema)
        fire_flush(u, rowsa, fsema)
        drain_gathers(u + 1, rowsb, gsemb)
        fire_flush(u + 1, rowsb, fsemb)
        drain_flush(u, rowsa, fsema)

        @pl.when(u + 2 < _UPW)
        def _():
            fire_gathers(u + 2, rowsa, gsema)

        drain_flush(u + 1, rowsb, fsemb)


def _transpose_body(in_ref, out_ref):
    x = in_ref[...]
    z = jnp.transpose(x, (1, 0))
    out_ref[...] = z.reshape(_PG, _NTI, 1, _TD, _TB)


_transpose_kernel = pl.pallas_call(
    _transpose_body,
    grid=(_NTJ, _NQ),
    in_specs=[pl.BlockSpec((_TB, 128), lambda tj, qh: (tj * _NQ + qh, 0))],
    out_specs=pl.BlockSpec((_PG, _NTI, 1, _TD, _TB),
                           lambda tj, qh: (qh, 0, tj, 0, 0)),
    out_shape=jax.ShapeDtypeStruct((_NP, _NTI, _NTJ, _TD, _TB),
                                   jnp.float32),
)


def kernel(species_ids, table):
    idxr = (species_ids.reshape(_NTJ, _TB, _NQ, _PG)
            .transpose(0, 2, 1, 3).reshape(-1).astype(jnp.int32))
    inter = _gather_kernel(idxr, table)
    in2d = inter.reshape(_B * _D // 128, 128)
    out5 = _transpose_kernel(in2d)
    return out5.transpose(2, 4, 0, 1, 3).reshape(_BATCH, _NP, _D)


# speedup vs baseline: 4.2160x; 2.3662x over previous
import functools

import jax
import jax.numpy as jnp
from jax import lax
from jax.experimental import pallas as pl
from jax.experimental.pallas import tpu as pltpu
from jax.experimental.pallas import tpu_sc as plsc

_BATCH = 16384
_NP = 20
_D = 32
_B = _BATCH * _NP
_NC = 2
_NS = 16
_NW = _NC * _NS
_BPW = _B // _NW
_GROW = 128
_GPC = 5
_CROW = _GROW * _GPC
_CB = _CROW // _NP
_NCHUNK = _BPW // _CROW
_BPWB = _BPW // _NP

_TB = 128
_TD = 8
_NTI = _D // _TD
_NTJ = _BATCH // _TB
_NQ = _NP * _D // 128

_mesh = plsc.VectorSubcoreMesh(
    core_axis_name="c", subcore_axis_name="s",
    num_cores=_NC, num_subcores=_NS)


@functools.partial(
    pl.kernel,
    out_type=jax.ShapeDtypeStruct((_BATCH, _NP, _D), jnp.float32),
    mesh=_mesh,
    compiler_params=pltpu.CompilerParams(use_tc_tiling_on_sc=False),
    scratch_types=[
        pltpu.VMEM((_BPW,), jnp.int32),
        pltpu.VMEM((_CROW, _D), jnp.float32),
        pltpu.VMEM((_CROW, _D), jnp.float32),
        pltpu.SemaphoreType.DMA,
        pltpu.SemaphoreType.DMA,
        pltpu.SemaphoreType.DMA,
        pltpu.SemaphoreType.DMA,
    ],
)
def _gather_kernel(idx_hbm, table_hbm, out_hbm, idx_v, rows0, rows1,
                   gsem0, gsem1, fsem0, fsem1):
    wid = lax.axis_index("s") * _NC + lax.axis_index("c")
    bbase = wid * _BPWB
    pltpu.sync_copy(idx_hbm.at[pl.ds(wid * _BPW, _BPW)], idx_v)

    def fire_gather(k, rows, sem):
        for g in range(_GPC):
            pltpu.async_copy(
                table_hbm.at[idx_v.at[pl.ds(k * _CROW + g * _GROW, _GROW)]],
                rows.at[pl.ds(g * _GROW, _GROW)], sem)

    def drain_gather(k, rows, sem):
        for g in range(_GPC):
            pltpu.make_async_copy(
                table_hbm.at[idx_v.at[pl.ds(k * _CROW + g * _GROW, _GROW)]],
                rows.at[pl.ds(g * _GROW, _GROW)], sem).wait()

    def fire_flush(k, rows, sem):
        for e in range(_CB):
            pltpu.async_copy(rows.at[pl.ds(e * _NP, _NP)],
                             out_hbm.at[bbase + k * _CB + e], sem)

    def drain_flush(k, rows, sem):
        for e in range(_CB):
            pltpu.make_async_copy(rows.at[pl.ds(e * _NP, _NP)],
                                  out_hbm.at[bbase + k * _CB + e],
                                  sem).wait()

    fire_gather(0, rows0, gsem0)

    @pl.loop(0, _NCHUNK, step=2)
    def _body(k):
        fire_gather(k + 1, rows1, gsem1)
        drain_gather(k, rows0, gsem0)
        fire_flush(k, rows0, fsem0)
        drain_gather(k + 1, rows1, gsem1)
        fire_flush(k + 1, rows1, fsem1)
        drain_flush(k, rows0, fsem0)

        @pl.when(k + 2 < _NCHUNK)
        def _():
            fire_gather(k + 2, rows0, gsem0)

        drain_flush(k + 1, rows1, fsem1)


def _transpose_body(in_ref, out_ref):
    x = in_ref[...]
    i2 = lax.broadcasted_iota(jnp.int32, (_CROW, _TB), 0)
    b2 = lax.broadcasted_iota(jnp.int32, (_CROW, _TB), 1)
    for j in range(_NQ):
        sel = (i2 == b2 * _NQ + j).astype(jnp.float32)
        z = lax.dot_general(x, sel, (((0,), (0,)), ((), ())),
                            preferred_element_type=jnp.float32)
        out_ref[j * 4:j * 4 + 4] = z.reshape(4, _NTI, 1, _TD, _TB)


_transpose_kernel = pl.pallas_call(
    _transpose_body,
    grid=(_NTJ,),
    in_specs=[pl.BlockSpec((_CROW, 128), lambda tj: (tj, 0))],
    out_specs=pl.BlockSpec((_NP, _NTI, 1, _TD, _TB),
                           lambda tj: (0, 0, tj, 0, 0)),
    out_shape=jax.ShapeDtypeStruct((_NP, _NTI, _NTJ, _TD, _TB),
                                   jnp.float32),
)


def kernel(species_ids, table):
    flat3 = _gather_kernel(species_ids.reshape(-1).astype(jnp.int32), table)
    in2d = flat3.reshape(_B * _D // 128, 128)
    out5 = _transpose_kernel(in2d)
    return out5.transpose(2, 4, 0, 1, 3).reshape(_BATCH, _NP, _D)
